# Initial kernel scaffold; baseline (speedup 1.0000x reference)
import jax
import jax.numpy as jnp
from jax.experimental import pallas as pl


def kernel(inputs, grid):
    raise NotImplementedError("write your pallas kernel here")


# speedup vs baseline: 6.6671x; 6.6671x over previous
import functools

import jax
import jax.numpy as jnp
from jax.experimental import pallas as pl

_M0 = 0.3
_K = 308
_N_TILE = 128


def _dtm_body(x_ref, g_ref, o_ref, *, k, weight_bound, n_iters):
    x = x_ref[0]
    x0 = x[:, 0:1]
    x1 = x[:, 1:2]
    g0 = g_ref[0:1, :]
    g1 = g_ref[1:2, :]
    dx = x0 - g0
    dy = x1 - g1
    d2 = dx * dx + dy * dy
    d2i = jax.lax.bitcast_convert_type(d2, jnp.int32)

    lo0 = jnp.zeros(g0.shape, jnp.int32)
    hi0 = jnp.full(g0.shape, 0x7F800000, jnp.int32)

    def step(_, carry):
        lo, hi = carry
        mid = lo + ((hi - lo) >> 1)
        cnt = jnp.sum((d2i <= mid).astype(jnp.int32), axis=0, keepdims=True)
        ge = cnt >= k
        return jnp.where(ge, lo, mid + 1), jnp.where(ge, mid, hi)

    lo, _ = jax.lax.fori_loop(0, n_iters, step, (lo0, hi0))
    t = jax.lax.bitcast_convert_type(lo, jnp.float32)

    less = d2 < t
    cnt_less = jnp.sum(less.astype(jnp.float32), axis=0, keepdims=True)
    sum_less = jnp.sum(jnp.where(less, d2, 0.0), axis=0, keepdims=True)
    dtm = jnp.sqrt((sum_less + (weight_bound - cnt_less) * t) / weight_bound)
    o_ref[0] = dtm


def kernel(inputs, grid):
    B, M, d = inputs.shape
    N = grid.shape[0]
    weight_bound = _M0 * M
    n_pad = pl.cdiv(N, _N_TILE) * _N_TILE

    gT = jnp.zeros((8, n_pad), jnp.float32)
    gT = gT.at[0, :N].set(grid[:, 0]).at[1, :N].set(grid[:, 1])

    body = functools.partial(
        _dtm_body, k=_K, weight_bound=weight_bound, n_iters=31)
    out = pl.pallas_call(
        body,
        grid=(B, n_pad // _N_TILE),
        in_specs=[
            pl.BlockSpec((1, M, d), lambda b, j: (b, 0, 0)),
            pl.BlockSpec((8, _N_TILE), lambda b, j: (0, j)),
        ],
        out_specs=pl.BlockSpec((1, 1, _N_TILE), lambda b, j: (b, 0, j)),
        out_shape=jax.ShapeDtypeStruct((B, 1, n_pad), jnp.float32),
    )(inputs, gT)
    return out[:, 0, :N]
